# baseline (speedup 1.0000x reference)
import functools

import jax
import jax.numpy as jnp
from jax import lax
from jax.experimental import pallas as pl
from jax.experimental.pallas import tpu as pltpu
from jax.experimental.pallas import tpu_sc as plsc

N_ROWS = 100000
N_FEAT = 8
N_TYPES = 128
N_WORKERS = 32
CHUNK = 160
N_CHUNKS = N_ROWS // CHUNK
N_SLOTS = -(-N_CHUNKS // N_WORKERS)
LAST_FULL_WID = N_CHUNKS - N_WORKERS * (N_SLOTS - 1)
N_OBUF = 4


def _make_kernel():
    mesh = plsc.VectorSubcoreMesh(core_axis_name="c", subcore_axis_name="s")

    @functools.partial(
        pl.kernel,
        mesh=mesh,
        compiler_params=pltpu.CompilerParams(needs_layout_passes=False),
        out_type=jax.ShapeDtypeStruct((N_ROWS, N_TYPES), jnp.float32),
        scratch_types=(
            [pltpu.VMEM((CHUNK,), jnp.int32) for _ in range(N_SLOTS)]
            + [pltpu.VMEM((CHUNK, N_TYPES), jnp.float32)] * N_OBUF
            + [pltpu.SemaphoreType.DMA] * (N_SLOTS + N_OBUF)
        ),
    )
    def onehot(idx_hbm, out_hbm, *scratch):
        ins = scratch[:N_SLOTS]
        bufs = scratch[N_SLOTS:N_SLOTS + N_OBUF]
        isems = scratch[N_SLOTS + N_OBUF:2 * N_SLOTS + N_OBUF]
        osems = scratch[2 * N_SLOTS + N_OBUF:]

        wid = lax.axis_index("s") * 2 + lax.axis_index("c")
        lanes = lax.iota(jnp.int32, 16)
        ones_f = jnp.full((16,), 1.0, jnp.float32)
        zeros_f = jnp.zeros((16,), jnp.float32)

        in_descs = []
        out_descs = []
        for t in range(N_SLOTS):
            base = (wid + t * N_WORKERS) * CHUNK
            in_descs.append(pltpu.make_async_copy(
                idx_hbm.at[pl.ds(base, CHUNK)], ins[t], isems[t]))
            out_descs.append(pltpu.make_async_copy(
                bufs[t % N_OBUF], out_hbm.at[pl.ds(base, CHUNK)],
                osems[t % N_OBUF]))

        for t in range(N_SLOTS - 1):
            in_descs[t].start()

        @pl.when(wid < LAST_FULL_WID)
        def _():
            in_descs[N_SLOTS - 1].start()

        def _zero_row(r, _):
            for buf in bufs:
                for c in range(N_TYPES // 16):
                    buf[r, pl.ds(c * 16, 16)] = zeros_f
            return 0

        lax.fori_loop(0, CHUNK, _zero_row, 0)

        for t in range(N_SLOTS):
            buf = bufs[t % N_OBUF]
            chunk = wid + t * N_WORKERS

            @pl.when(chunk < N_CHUNKS)
            def _(t=t, buf=buf):
                in_descs[t].wait()
                if t >= N_OBUF:
                    out_descs[t - N_OBUF].wait()
                    for g in range(CHUNK // 16):
                        rows = lanes + (g * 16)
                        old_cols = ins[t - N_OBUF][pl.ds(g * 16, 16)]
                        plsc.store_scatter(buf, [rows, old_cols], zeros_f)
                for g in range(CHUNK // 16):
                    rows = lanes + (g * 16)
                    cols = ins[t][pl.ds(g * 16, 16)]
                    plsc.store_scatter(buf, [rows, cols], ones_f)
                pltpu.touch(buf)
                out_descs[t].start()

        for t in range(max(0, N_SLOTS - N_OBUF - 1), N_SLOTS):
            chunk = wid + t * N_WORKERS

            @pl.when((chunk < N_CHUNKS)
                     & (chunk + N_OBUF * N_WORKERS >= N_CHUNKS))
            def _(t=t):
                out_descs[t].wait()

    return onehot


_onehot = _make_kernel()


@jax.jit
def kernel(node_feat):
    idx = node_feat[:, 0].astype(jnp.int32)
    return _onehot(idx)


# speedup vs baseline: 1.0533x; 1.0533x over previous
import functools

import jax
import jax.numpy as jnp
from jax import lax
from jax.experimental import pallas as pl
from jax.experimental.pallas import tpu as pltpu
from jax.experimental.pallas import tpu_sc as plsc

N_ROWS = 100000
N_FEAT = 8
N_TYPES = 128
N_WORKERS = 32
CHUNK = 160
N_CHUNKS = N_ROWS // CHUNK
N_SLOTS = -(-N_CHUNKS // N_WORKERS)
LAST_FULL_WID = N_CHUNKS - N_WORKERS * (N_SLOTS - 1)
N_OBUF = 2


def _make_kernel():
    mesh = plsc.VectorSubcoreMesh(core_axis_name="c", subcore_axis_name="s")

    @functools.partial(
        pl.kernel,
        mesh=mesh,
        compiler_params=pltpu.CompilerParams(needs_layout_passes=False),
        out_type=jax.ShapeDtypeStruct((N_ROWS, N_TYPES), jnp.float32),
        scratch_types=(
            [pltpu.VMEM((CHUNK,), jnp.int32) for _ in range(N_SLOTS)]
            + [pltpu.VMEM((CHUNK, N_TYPES), jnp.float32)] * N_OBUF
            + [pltpu.SemaphoreType.DMA] * (N_SLOTS + N_OBUF)
        ),
    )
    def onehot(idx_hbm, out_hbm, *scratch):
        ins = scratch[:N_SLOTS]
        bufs = scratch[N_SLOTS:N_SLOTS + N_OBUF]
        isems = scratch[N_SLOTS + N_OBUF:2 * N_SLOTS + N_OBUF]
        osems = scratch[2 * N_SLOTS + N_OBUF:]

        wid = lax.axis_index("s") * 2 + lax.axis_index("c")
        lanes = lax.iota(jnp.int32, 16)
        ones_f = jnp.full((16,), 1.0, jnp.float32)
        zeros_f = jnp.zeros((16,), jnp.float32)

        in_descs = []
        out_descs = []
        for t in range(N_SLOTS):
            base = (wid + t * N_WORKERS) * CHUNK
            in_descs.append(pltpu.make_async_copy(
                idx_hbm.at[pl.ds(base, CHUNK)], ins[t], isems[t]))
            out_descs.append(pltpu.make_async_copy(
                bufs[t % N_OBUF], out_hbm.at[pl.ds(base, CHUNK)],
                osems[t % N_OBUF]))

        for t in range(N_SLOTS - 1):
            in_descs[t].start()

        @pl.when(wid < LAST_FULL_WID)
        def _():
            in_descs[N_SLOTS - 1].start()

        def _zero_row(r, _):
            for buf in bufs:
                for c in range(N_TYPES // 16):
                    buf[r, pl.ds(c * 16, 16)] = zeros_f
            return 0

        lax.fori_loop(0, CHUNK, _zero_row, 0)

        for t in range(N_SLOTS):
            buf = bufs[t % N_OBUF]
            chunk = wid + t * N_WORKERS

            @pl.when(chunk < N_CHUNKS)
            def _(t=t, buf=buf):
                in_descs[t].wait()
                if t >= N_OBUF:
                    out_descs[t - N_OBUF].wait()
                    for g in range(CHUNK // 16):
                        rows = lanes + (g * 16)
                        old_cols = ins[t - N_OBUF][pl.ds(g * 16, 16)]
                        plsc.store_scatter(buf, [rows, old_cols], zeros_f)
                for g in range(CHUNK // 16):
                    rows = lanes + (g * 16)
                    cols = ins[t][pl.ds(g * 16, 16)]
                    plsc.store_scatter(buf, [rows, cols], ones_f)
                pltpu.touch(buf)
                out_descs[t].start()

        for t in range(max(0, N_SLOTS - N_OBUF - 1), N_SLOTS):
            chunk = wid + t * N_WORKERS

            @pl.when((chunk < N_CHUNKS)
                     & (chunk + N_OBUF * N_WORKERS >= N_CHUNKS))
            def _(t=t):
                out_descs[t].wait()

    return onehot


_onehot = _make_kernel()


@jax.jit
def kernel(node_feat):
    idx = node_feat[:, 0].astype(jnp.int32)
    return _onehot(idx)
